# baseline (speedup 1.0000x reference)
import jax
import jax.numpy as jnp
import numpy as np
from jax.experimental import pallas as pl

_EMB_LIST = [100, 11, 11, 11, 9, 4, 9, 5, 4, 8]
_TOTAL = 172
_NFEAT = 10
_GROUP = 32
_WIDE = _GROUP * _TOTAL
_KDIM = _GROUP * _NFEAT

_OFFSETS = np.concatenate([[0], np.cumsum(_EMB_LIST)[:-1]])
_FEAT_OF_COL = np.repeat(np.arange(_NFEAT), _EMB_LIST)
_LOCAL_OF_COL = np.arange(_TOTAL) - _OFFSETS[_FEAT_OF_COL]

_Q = np.arange(_WIDE)
_K_OF_Q = (_Q // _TOTAL) * _NFEAT + _FEAT_OF_COL[_Q % _TOTAL]
_GATHER_MAT = (np.arange(_KDIM)[:, None] == _K_OF_Q[None, :]).astype(
    np.float32
)
_LOCAL_WIDE = np.tile(_LOCAL_OF_COL, _GROUP).astype(np.float32)


def _onehot_block(atom_ref, gmat_ref, local_ref, out_ref):
    a = atom_ref[0].astype(jnp.bfloat16)
    g = jax.lax.dot_general(
        a,
        gmat_ref[...],
        (((1,), (0,)), ((), ())),
        preferred_element_type=jnp.float32,
    )
    out_ref[0] = jnp.where(g == local_ref[...], 1.0, 0.0)


def _wide_kernel(atom_flat, block_srows):
    s = atom_flat.shape[0]
    grid = s // block_srows
    atom3 = atom_flat.reshape(grid, block_srows, _KDIM)
    gmat = jnp.asarray(_GATHER_MAT, dtype=jnp.bfloat16)
    local = jnp.asarray(_LOCAL_WIDE)[None, :]
    out = pl.pallas_call(
        _onehot_block,
        out_shape=jax.ShapeDtypeStruct((grid, block_srows, _WIDE), jnp.float32),
        grid=(grid,),
        in_specs=[
            pl.BlockSpec((1, block_srows, _KDIM), lambda i: (i, 0, 0)),
            pl.BlockSpec((_KDIM, _WIDE), lambda i: (0, 0)),
            pl.BlockSpec((1, _WIDE), lambda i: (0, 0)),
        ],
        out_specs=pl.BlockSpec((1, block_srows, _WIDE), lambda i: (i, 0, 0)),
    )(atom3, gmat, local)
    return out


@jax.jit
def kernel(atom):
    n = atom.shape[0]
    atom_flat = atom.astype(jnp.int32).reshape(n // _GROUP, _KDIM)
    wide = _wide_kernel(atom_flat, 125)
    return wide.reshape(n, _TOTAL)
